# baseline (speedup 1.0000x reference)
import functools

import jax
import jax.numpy as jnp
from jax import lax
from jax.experimental import pallas as pl
from jax.experimental.pallas import tpu as pltpu
from jax.experimental.pallas import tpu_sc as plsc

_CH = 128


@functools.lru_cache(maxsize=None)
def _make_sc_gather(B):
    info = plsc.get_sparse_core_info()
    NW = info.num_cores * info.num_subcores
    NC = info.num_cores
    assert B % (NW * _CH) == 0
    n_ch = B // (NW * _CH)
    rows = B // _CH

    mesh = plsc.VectorSubcoreMesh(core_axis_name="c", subcore_axis_name="s")

    @functools.partial(
        pl.kernel,
        mesh=mesh,
        out_type=jax.ShapeDtypeStruct((rows, _CH), jnp.float32),
        scratch_types=[
            pltpu.VMEM((n_ch, _CH), jnp.int32),
            pltpu.VMEM((n_ch, _CH), jnp.float32),
            pltpu.SemaphoreType.DMA,
        ],
    )
    def gather_k(table_hbm, idx_hbm, out_hbm, idx_v, vals_v, sem):
        wid = lax.axis_index("s") * NC + lax.axis_index("c")
        base = wid * n_ch
        pltpu.sync_copy(idx_hbm.at[pl.ds(base, n_ch)], idx_v)
        copies = []
        for j in range(n_ch):
            copies.append(
                pltpu.async_copy(table_hbm.at[idx_v.at[j]], vals_v.at[j], sem)
            )
        for c in copies:
            c.wait()
        pltpu.sync_copy(vals_v, out_hbm.at[pl.ds(base, n_ch)])

    return gather_k


@functools.lru_cache(maxsize=None)
def _make_sc_full(n_users, B):
    info = plsc.get_sparse_core_info()
    NW = info.num_cores * info.num_subcores
    NC = info.num_cores
    cols = B // NW
    n_ch = cols // _CH
    RR = 32
    n_wr = n_users // RR

    mesh = plsc.VectorSubcoreMesh(core_axis_name="c", subcore_axis_name="s")

    @functools.partial(
        pl.kernel,
        mesh=mesh,
        out_type=jax.ShapeDtypeStruct((n_users, B), jnp.float32),
        scratch_types=[
            pltpu.VMEM((n_ch, _CH), jnp.int32),
            pltpu.VMEM((RR, cols), jnp.float32),
            pltpu.SemaphoreType.DMA,
            pltpu.SemaphoreType.DMA,
        ],
    )
    def full_k(table_hbm, idx_hbm, out_hbm, idx_v, buf_v, sem, wsem):
        wid = lax.axis_index("s") * NC + lax.axis_index("c")
        base = wid * n_ch
        pltpu.sync_copy(idx_hbm.at[pl.ds(base, n_ch)], idx_v)
        gathers = [
            pltpu.async_copy(
                table_hbm.at[idx_v.at[j]],
                buf_v.at[0, pl.ds(j * _CH, _CH)],
                sem,
            )
            for j in range(n_ch)
        ]
        for g in gathers:
            g.wait()
        regs = [buf_v[0, pl.ds(c * 16, 16)] for c in range(cols // 16)]
        for r in range(1, RR):
            for c, reg in enumerate(regs):
                buf_v[r, pl.ds(c * 16, 16)] = reg
        col0 = wid * cols
        writes = [
            pltpu.async_copy(
                buf_v,
                out_hbm.at[pl.ds(i * RR, RR), pl.ds(col0, cols)],
                wsem,
            )
            for i in range(n_wr)
        ]
        for w in writes:
            w.wait()

    return full_k


_RB = 32


@functools.lru_cache(maxsize=None)
def _make_bcast(n_users, B):
    n_dma = n_users // _RB

    def _bcast_body(r_ref, o_ref, buf, sem):
        buf[...] = jnp.broadcast_to(r_ref[...], buf.shape)
        copies = [
            pltpu.make_async_copy(buf, o_ref.at[pl.ds(i * _RB, _RB), :], sem)
            for i in range(n_dma)
        ]
        for c in copies:
            c.start()
        for c in copies:
            c.wait()

    return pl.pallas_call(
        _bcast_body,
        in_specs=[pl.BlockSpec(memory_space=pltpu.VMEM)],
        out_specs=pl.BlockSpec(memory_space=pl.ANY),
        out_shape=jax.ShapeDtypeStruct((n_users, B), jnp.float32),
        scratch_shapes=[
            pltpu.VMEM((_RB, B), jnp.float32),
            pltpu.SemaphoreType.DMA,
        ],
    )


def kernel(user_ids, item_ids, items_count):
    n_users = user_ids.shape[0]
    B = item_ids.shape[0]
    idx2d = item_ids.reshape(-1, _CH)
    return _make_sc_full(n_users, B)(items_count, idx2d)


# speedup vs baseline: 1.0256x; 1.0256x over previous
import functools

import jax
import jax.numpy as jnp
from jax import lax
from jax.experimental import pallas as pl
from jax.experimental.pallas import tpu as pltpu
from jax.experimental.pallas import tpu_sc as plsc

_CH = 128


@functools.lru_cache(maxsize=None)
def _make_sc_gather(B, num_cores=None):
    info = plsc.get_sparse_core_info()
    NC = num_cores if num_cores is not None else info.num_cores
    NW = NC * info.num_subcores
    assert B % (NW * _CH) == 0
    n_ch = B // (NW * _CH)
    rows = B // _CH

    if num_cores is not None:
        mesh = plsc.VectorSubcoreMesh(
            core_axis_name="c", subcore_axis_name="s", num_cores=NC
        )
    else:
        mesh = plsc.VectorSubcoreMesh(core_axis_name="c", subcore_axis_name="s")

    @functools.partial(
        pl.kernel,
        mesh=mesh,
        out_type=jax.ShapeDtypeStruct((rows, _CH), jnp.float32),
        scratch_types=[
            pltpu.VMEM((n_ch, _CH), jnp.int32),
            pltpu.VMEM((n_ch, _CH), jnp.float32),
            pltpu.SemaphoreType.DMA,
        ],
    )
    def gather_k(table_hbm, idx_hbm, out_hbm, idx_v, vals_v, sem):
        wid = lax.axis_index("s") * NC + lax.axis_index("c")
        base = wid * n_ch
        pltpu.sync_copy(idx_hbm.at[pl.ds(base, n_ch)], idx_v)
        copies = []
        for j in range(n_ch):
            copies.append(
                pltpu.async_copy(table_hbm.at[idx_v.at[j]], vals_v.at[j], sem)
            )
        for c in copies:
            c.wait()
        pltpu.sync_copy(vals_v, out_hbm.at[pl.ds(base, n_ch)])

    return gather_k


@functools.lru_cache(maxsize=None)
def _make_sc_min():
    mesh = plsc.VectorSubcoreMesh(core_axis_name="c", subcore_axis_name="s")

    @functools.partial(
        pl.kernel,
        mesh=mesh,
        out_type=jax.ShapeDtypeStruct((1, _CH), jnp.float32),
        scratch_types=[pltpu.VMEM((1, _CH), jnp.float32)],
    )
    def min_k(table_hbm, out_hbm, buf_v):
        wid = lax.axis_index("s") * 2 + lax.axis_index("c")

        @pl.when(wid == 0)
        def _():
            pltpu.sync_copy(table_hbm.at[pl.ds(0, 1)], buf_v)
            pltpu.sync_copy(buf_v, out_hbm)

    return min_k


@functools.lru_cache(maxsize=None)
def _make_sc_full(n_users, B):
    info = plsc.get_sparse_core_info()
    NW = info.num_cores * info.num_subcores
    NC = info.num_cores
    cols = B // NW
    n_ch = cols // _CH
    RR = 32
    n_wr = n_users // RR

    mesh = plsc.VectorSubcoreMesh(core_axis_name="c", subcore_axis_name="s")

    @functools.partial(
        pl.kernel,
        mesh=mesh,
        out_type=jax.ShapeDtypeStruct((n_users, B), jnp.float32),
        scratch_types=[
            pltpu.VMEM((n_ch, _CH), jnp.int32),
            pltpu.VMEM((RR, cols), jnp.float32),
            pltpu.SemaphoreType.DMA,
            pltpu.SemaphoreType.DMA,
        ],
    )
    def full_k(table_hbm, idx_hbm, out_hbm, idx_v, buf_v, sem, wsem):
        wid = lax.axis_index("s") * NC + lax.axis_index("c")
        base = wid * n_ch
        pltpu.sync_copy(idx_hbm.at[pl.ds(base, n_ch)], idx_v)
        gathers = [
            pltpu.async_copy(
                table_hbm.at[idx_v.at[j]],
                buf_v.at[0, pl.ds(j * _CH, _CH)],
                sem,
            )
            for j in range(n_ch)
        ]
        for g in gathers:
            g.wait()
        regs = [buf_v[0, pl.ds(c * 16, 16)] for c in range(cols // 16)]
        for r in range(1, RR):
            for c, reg in enumerate(regs):
                buf_v[r, pl.ds(c * 16, 16)] = reg
        col0 = wid * cols
        writes = [
            pltpu.async_copy(
                buf_v,
                out_hbm.at[pl.ds(i * RR, RR), pl.ds(col0, cols)],
                wsem,
            )
            for i in range(n_wr)
        ]
        for w in writes:
            w.wait()

    return full_k


_RB = 8


@functools.lru_cache(maxsize=None)
def _make_bcast(n_users, B):
    n_dma = n_users // _RB

    def _bcast_body(r_ref, o_ref, buf, sem):
        buf[...] = jnp.broadcast_to(r_ref[...], buf.shape)
        copies = [
            pltpu.make_async_copy(buf, o_ref.at[pl.ds(i * _RB, _RB), :], sem)
            for i in range(n_dma)
        ]
        for c in copies:
            c.start()
        for c in copies:
            c.wait()

    return pl.pallas_call(
        _bcast_body,
        in_specs=[pl.BlockSpec(memory_space=pltpu.VMEM)],
        out_specs=pl.BlockSpec(memory_space=pl.ANY),
        out_shape=jax.ShapeDtypeStruct((n_users, B), jnp.float32),
        scratch_shapes=[
            pltpu.VMEM((_RB, B), jnp.float32),
            pltpu.SemaphoreType.DMA,
        ],
    )


def kernel(user_ids, item_ids, items_count):
    n_users = user_ids.shape[0]
    B = item_ids.shape[0]
    idx2d = item_ids.reshape(-1, _CH)
    ratings = _make_sc_gather(B)(items_count, idx2d)
    return _make_bcast(n_users, B)(ratings.reshape(1, B))
